# baseline (speedup 1.0000x reference)
import jax
import jax.numpy as jnp
from jax.experimental import pallas as pl
from jax.experimental.pallas import tpu as pltpu

_BI = 200


def _fused_body(a_ref, h_ref, w_ref, ak1_ref, ak2_ref, b_ref, o_ref,
                acc_ref, zc_ref, hw_ref, c_ref, q1_ref, q2_ref):
    i = pl.program_id(0)
    ni = pl.num_programs(0)
    bi, n = a_ref.shape

    @pl.when(i == 0)
    def _():
        hw = jnp.dot(h_ref[...], w_ref[...],
                     preferred_element_type=jnp.float32)
        hw_ref[...] = hw
        a1 = jnp.dot(hw, ak1_ref[...],
                     preferred_element_type=jnp.float32)
        hwt = jnp.transpose(hw)
        a2r = jnp.sum(hwt * ak2_ref[...], axis=0, keepdims=True)
        g = jnp.max(a2r)
        c_ref[...] = jnp.minimum(jnp.exp(-0.8 * (a1 + g)),
                                 1e30).astype(jnp.bfloat16)
        q1_ref[...] = jnp.exp(a2r - g).astype(jnp.bfloat16)
        q2_ref[...] = jnp.exp(0.2 * (a2r - g)).astype(jnp.bfloat16)

    row = pl.multiple_of(i * bi, 8)
    ab = a_ref[...].astype(jnp.bfloat16)
    cs = c_ref[pl.ds(row, bi), :]
    ad = ab * jnp.maximum(q1_ref[...], cs * q2_ref[...])
    s = jnp.sum(ad.astype(jnp.float32), axis=1, keepdims=True)
    pos = s > 0.0
    r = jnp.where(pos, 1.0 / jnp.where(pos, s, 1.0), 0.0)
    z = jnp.where(pos, 0.0, 1.0 / n)
    hwt = jnp.transpose(hw_ref[pl.ds(row, bi), :])
    hwtr = (hwt * jnp.transpose(r)).astype(jnp.bfloat16)
    part = jax.lax.dot_general(
        hwtr, ad, (((1,), (0,)), ((), ())),
        preferred_element_type=jnp.float32)
    zpart = jax.lax.dot_general(
        hwt, z, (((1,), (0,)), ((), ())),
        preferred_element_type=jnp.float32)

    @pl.when(i == 0)
    def _():
        acc_ref[...] = part
        zc_ref[...] = zpart

    @pl.when(i != 0)
    def _():
        acc_ref[...] += part
        zc_ref[...] += zpart

    @pl.when(i == ni - 1)
    def _():
        res_t = jnp.maximum(acc_ref[...] + (b_ref[...] + zc_ref[...]), 0.0)
        o_ref[...] = jnp.transpose(res_t)


@jax.jit
def kernel(H, A, idx, kernel, bias, attn_kernel_1, attn_kernel_2):
    del idx
    n, f_in = H.shape
    f_out = kernel.shape[1]
    bi = _BI

    out = pl.pallas_call(
        _fused_body,
        grid=(n // bi,),
        in_specs=[
            pl.BlockSpec((bi, n), lambda i: (i, 0)),
            pl.BlockSpec((n, f_in), lambda i: (0, 0)),
            pl.BlockSpec((f_in, f_out), lambda i: (0, 0)),
            pl.BlockSpec((f_out, 1), lambda i: (0, 0)),
            pl.BlockSpec((f_out, 1), lambda i: (0, 0)),
            pl.BlockSpec((f_out, 1), lambda i: (0, 0)),
        ],
        out_specs=pl.BlockSpec((n, f_out), lambda i: (0, 0)),
        out_shape=jax.ShapeDtypeStruct((n, f_out), jnp.float32),
        scratch_shapes=[
            pltpu.VMEM((f_out, n), jnp.float32),
            pltpu.VMEM((f_out, 1), jnp.float32),
            pltpu.VMEM((n, f_out), jnp.float32),
            pltpu.VMEM((n, 1), jnp.bfloat16),
            pltpu.VMEM((1, n), jnp.bfloat16),
            pltpu.VMEM((1, n), jnp.bfloat16),
        ],
    )(A, H, kernel, attn_kernel_1, attn_kernel_2, bias.reshape(f_out, 1))
    return out


# speedup vs baseline: 1.0655x; 1.0655x over previous
import jax
import jax.numpy as jnp
from jax.experimental import pallas as pl
from jax.experimental.pallas import tpu as pltpu

_BI = 400


def _setup_body(h_ref, w_ref, ak1_ref, ak2_ref, hw_ref, c_ref, q1_ref,
                q2_ref):
    hw = jnp.dot(h_ref[...], w_ref[...], preferred_element_type=jnp.float32)
    hw_ref[...] = hw
    a1 = jnp.dot(hw, ak1_ref[...], preferred_element_type=jnp.float32)
    hwt = jnp.transpose(hw)
    a2r = jnp.sum(hwt * ak2_ref[...], axis=0, keepdims=True)
    g = jnp.max(a2r)
    c_ref[...] = jnp.minimum(jnp.exp(-0.8 * (a1 + g)), 1e30).astype(
        jnp.bfloat16)
    q1_ref[...] = jnp.exp(a2r - g).astype(jnp.bfloat16)
    q2_ref[...] = jnp.exp(0.2 * (a2r - g)).astype(jnp.bfloat16)


def _fused_body(a_ref, c_ref, q1_ref, q2_ref, hw_ref, b_ref, o_ref,
                acc_ref, zc_ref):
    i = pl.program_id(0)
    ni = pl.num_programs(0)
    n = a_ref.shape[1]
    ab = a_ref[...].astype(jnp.bfloat16)
    ad = ab * jnp.maximum(q1_ref[...], c_ref[...] * q2_ref[...])
    s = jnp.sum(ad.astype(jnp.float32), axis=1, keepdims=True)
    pos = s > 0.0
    r = jnp.where(pos, 1.0 / jnp.where(pos, s, 1.0), 0.0)
    z = jnp.where(pos, 0.0, 1.0 / n)
    hwt = jnp.transpose(hw_ref[...])
    hwtr = (hwt * jnp.transpose(r)).astype(jnp.bfloat16)
    part = jax.lax.dot_general(
        hwtr, ad, (((1,), (0,)), ((), ())),
        preferred_element_type=jnp.float32)
    zpart = jax.lax.dot_general(
        hwt, z, (((1,), (0,)), ((), ())),
        preferred_element_type=jnp.float32)

    @pl.when(i == 0)
    def _():
        acc_ref[...] = part
        zc_ref[...] = zpart

    @pl.when(i != 0)
    def _():
        acc_ref[...] += part
        zc_ref[...] += zpart

    @pl.when(i == ni - 1)
    def _():
        res_t = jnp.maximum(acc_ref[...] + (b_ref[...] + zc_ref[...]), 0.0)
        o_ref[...] = jnp.transpose(res_t)


@jax.jit
def kernel(H, A, idx, kernel, bias, attn_kernel_1, attn_kernel_2):
    del idx
    n, f_in = H.shape
    f_out = kernel.shape[1]

    hw, c, q1, q2 = pl.pallas_call(
        _setup_body,
        grid=(1,),
        in_specs=[
            pl.BlockSpec((n, f_in), lambda i: (0, 0)),
            pl.BlockSpec((f_in, f_out), lambda i: (0, 0)),
            pl.BlockSpec((f_out, 1), lambda i: (0, 0)),
            pl.BlockSpec((f_out, 1), lambda i: (0, 0)),
        ],
        out_specs=[
            pl.BlockSpec((n, f_out), lambda i: (0, 0)),
            pl.BlockSpec((n, 1), lambda i: (0, 0)),
            pl.BlockSpec((1, n), lambda i: (0, 0)),
            pl.BlockSpec((1, n), lambda i: (0, 0)),
        ],
        out_shape=[
            jax.ShapeDtypeStruct((n, f_out), jnp.float32),
            jax.ShapeDtypeStruct((n, 1), jnp.bfloat16),
            jax.ShapeDtypeStruct((1, n), jnp.bfloat16),
            jax.ShapeDtypeStruct((1, n), jnp.bfloat16),
        ],
    )(H, kernel, attn_kernel_1, attn_kernel_2)

    bi = _BI
    out = pl.pallas_call(
        _fused_body,
        grid=(n // bi,),
        in_specs=[
            pl.BlockSpec((bi, n), lambda i: (i, 0)),
            pl.BlockSpec((bi, 1), lambda i: (i, 0)),
            pl.BlockSpec((1, n), lambda i: (0, 0)),
            pl.BlockSpec((1, n), lambda i: (0, 0)),
            pl.BlockSpec((bi, f_out), lambda i: (i, 0)),
            pl.BlockSpec((f_out, 1), lambda i: (0, 0)),
        ],
        out_specs=pl.BlockSpec((n, f_out), lambda i: (0, 0)),
        out_shape=jax.ShapeDtypeStruct((n, f_out), jnp.float32),
        scratch_shapes=[
            pltpu.VMEM((f_out, n), jnp.float32),
            pltpu.VMEM((f_out, 1), jnp.float32),
        ],
    )(A, c, q1, q2, hw, bias.reshape(f_out, 1))
    return out


# speedup vs baseline: 1.0969x; 1.0294x over previous
import jax
import jax.numpy as jnp
from jax.experimental import pallas as pl
from jax.experimental.pallas import tpu as pltpu

_BI = 400


def _fused_body(a_ref, h_ref, w_ref, ak1_ref, ak2_ref, b_ref, o_ref,
                acc_ref, zc_ref, hw_ref, c_ref, q1_ref, q2_ref):
    i = pl.program_id(0)
    ni = pl.num_programs(0)
    bi, n = a_ref.shape

    @pl.when(i == 0)
    def _():
        wb = w_ref[...].astype(jnp.bfloat16)
        for k in range(n // bi):
            hw_ref[k * bi:(k + 1) * bi, :] = jnp.dot(
                h_ref[k * bi:(k + 1) * bi, :], wb,
                preferred_element_type=jnp.float32).astype(jnp.bfloat16)
        hwt = jnp.transpose(hw_ref[...])
        a2r = jnp.sum(hwt.astype(jnp.float32) * ak2_ref[...],
                      axis=0, keepdims=True)
        g = jnp.max(a2r)
        q1_ref[...] = jnp.exp(a2r - g).astype(jnp.bfloat16)
        q2_ref[...] = jnp.exp(0.2 * (a2r - g)).astype(jnp.bfloat16)
        for k in range(n // bi):
            hk = jnp.dot(hw_ref[k * bi:(k + 1) * bi, :].astype(jnp.float32),
                         ak1_ref[...],
                         preferred_element_type=jnp.float32)
            c_ref[k * bi:(k + 1) * bi, :] = jnp.minimum(
                jnp.exp(-0.8 * (hk + g)), 1e30).astype(jnp.bfloat16)

    row = pl.multiple_of(i * bi, 8)
    ab = a_ref[...].astype(jnp.bfloat16)
    cs = c_ref[pl.ds(row, bi), :]
    ad = ab * jnp.maximum(q1_ref[...], cs * q2_ref[...])
    s = jnp.sum(ad.astype(jnp.float32), axis=1, keepdims=True)
    pos = s > 0.0
    r = jnp.where(pos, 1.0 / jnp.where(pos, s, 1.0), 0.0)
    z = jnp.where(pos, 0.0, 1.0 / n)
    hwt = jnp.transpose(hw_ref[pl.ds(row, bi), :])
    hwtr = hwt * jnp.transpose(r).astype(jnp.bfloat16)
    part = jax.lax.dot_general(
        hwtr, ad, (((1,), (0,)), ((), ())),
        preferred_element_type=jnp.float32)
    zpart = jax.lax.dot_general(
        hwt.astype(jnp.float32), z, (((1,), (0,)), ((), ())),
        preferred_element_type=jnp.float32)

    @pl.when(i == 0)
    def _():
        acc_ref[...] = part
        zc_ref[...] = zpart

    @pl.when(i != 0)
    def _():
        acc_ref[...] += part
        zc_ref[...] += zpart

    @pl.when(i == ni - 1)
    def _():
        res_t = jnp.maximum(acc_ref[...] + (b_ref[...] + zc_ref[...]), 0.0)
        o_ref[...] = jnp.transpose(res_t)


@jax.jit
def kernel(H, A, idx, kernel, bias, attn_kernel_1, attn_kernel_2):
    del idx
    n, f_in = H.shape
    f_out = kernel.shape[1]
    bi = _BI

    out = pl.pallas_call(
        _fused_body,
        grid=(n // bi,),
        in_specs=[
            pl.BlockSpec((bi, n), lambda i: (i, 0)),
            pl.BlockSpec((n, f_in), lambda i: (0, 0)),
            pl.BlockSpec((f_in, f_out), lambda i: (0, 0)),
            pl.BlockSpec((f_out, 1), lambda i: (0, 0)),
            pl.BlockSpec((f_out, 1), lambda i: (0, 0)),
            pl.BlockSpec((f_out, 1), lambda i: (0, 0)),
        ],
        out_specs=pl.BlockSpec((n, f_out), lambda i: (0, 0)),
        out_shape=jax.ShapeDtypeStruct((n, f_out), jnp.float32),
        scratch_shapes=[
            pltpu.VMEM((f_out, n), jnp.float32),
            pltpu.VMEM((f_out, 1), jnp.float32),
            pltpu.VMEM((n, f_out), jnp.bfloat16),
            pltpu.VMEM((n, 1), jnp.bfloat16),
            pltpu.VMEM((1, n), jnp.bfloat16),
            pltpu.VMEM((1, n), jnp.bfloat16),
        ],
    )(A, H.astype(jnp.bfloat16), kernel, attn_kernel_1, attn_kernel_2,
      bias.reshape(f_out, 1))
    return out
